# Initial kernel scaffold; baseline (speedup 1.0000x reference)
import jax
import jax.numpy as jnp
from jax.experimental import pallas as pl


def kernel(feat_user, feat_item, edge_follows, edge_rates, edge_rated_by, W_follows, W_rates, W_rated_by, ln_gamma, ln_beta):
    raise NotImplementedError("write your pallas kernel here")


# speedup vs baseline: 1.0090x; 1.0090x over previous
import jax
import jax.numpy as jnp
from jax.experimental import pallas as pl

_N = 50000
_D = 128
_BLK = 1000


def _transform_kernel(fu_ref, fi_ref, wc_ref, wrb_ref, tu_ref, ti_ref):
    tu_ref[...] = jnp.dot(fu_ref[...], wc_ref[...],
                          preferred_element_type=jnp.float32)
    ti_ref[...] = jnp.dot(fi_ref[...], wrb_ref[...],
                          preferred_element_type=jnp.float32)


def _finish_kernel(sf_ref, cf_ref, srb_ref, crb_ref, sr_ref, cr_ref,
                   g_ref, b_ref, ou_ref, oi_ref):
    eps = 1e-5
    agg_f = sf_ref[...] / jnp.maximum(cf_ref[...], 1.0)
    agg_rb = srb_ref[...] / jnp.maximum(crb_ref[...], 1.0)
    h_u = (agg_f + agg_rb) * 0.5
    h_i = sr_ref[...] / jnp.maximum(cr_ref[...], 1.0)

    def _ln(x):
        x = jnp.maximum(x, 0.0)
        mu = jnp.mean(x, axis=-1, keepdims=True)
        var = jnp.mean(jnp.square(x - mu), axis=-1, keepdims=True)
        return (x - mu) * jax.lax.rsqrt(var + eps) * g_ref[...] + b_ref[...]

    ou_ref[...] = _ln(h_u)
    oi_ref[...] = _ln(h_i)


def _seg_sum_cnt(t_src, src, dst):
    msg = jnp.take(t_src, src, axis=0)
    s = jax.ops.segment_sum(msg, dst, num_segments=_N)
    cnt = jax.ops.segment_sum(jnp.ones((src.shape[0], 1), dtype=t_src.dtype),
                              dst, num_segments=_N)
    return s, cnt


def kernel(feat_user, feat_item, edge_follows, edge_rates, edge_rated_by,
           W_follows, W_rates, W_rated_by, ln_gamma, ln_beta):
    w_comb = ((W_follows + W_rates) * 0.5).T
    w_rb = W_rated_by.T

    grid = _N // _BLK
    row_spec = pl.BlockSpec((_BLK, _D), lambda i: (i, 0))
    mat_spec = pl.BlockSpec((_D, _D), lambda i: (0, 0))
    t_user, t_item = pl.pallas_call(
        _transform_kernel,
        grid=(grid,),
        in_specs=[row_spec, row_spec, mat_spec, mat_spec],
        out_specs=[row_spec, row_spec],
        out_shape=[jax.ShapeDtypeStruct((_N, _D), jnp.float32)] * 2,
    )(feat_user, feat_item, w_comb, w_rb)

    s_f, c_f = _seg_sum_cnt(t_user, edge_follows[0], edge_follows[1])
    s_rb, c_rb = _seg_sum_cnt(t_item, edge_rated_by[0], edge_rated_by[1])
    s_r, c_r = _seg_sum_cnt(t_user, edge_rates[0], edge_rates[1])

    cnt_spec = pl.BlockSpec((_BLK, 1), lambda i: (i, 0))
    vec_spec = pl.BlockSpec((1, _D), lambda i: (0, 0))
    out_user, out_item = pl.pallas_call(
        _finish_kernel,
        grid=(grid,),
        in_specs=[row_spec, cnt_spec, row_spec, cnt_spec, row_spec, cnt_spec,
                  vec_spec, vec_spec],
        out_specs=[row_spec, row_spec],
        out_shape=[jax.ShapeDtypeStruct((_N, _D), jnp.float32)] * 2,
    )(s_f, c_f, s_rb, c_rb, s_r, c_r,
      ln_gamma.reshape(1, _D), ln_beta.reshape(1, _D))

    return (out_user, out_item)
